# baseline (speedup 1.0000x reference)
import functools

import jax
import jax.numpy as jnp
from jax import lax
from jax.experimental import pallas as pl
from jax.experimental.pallas import tpu as pltpu
from jax.experimental.pallas import tpu_sc as plsc

N = 10000
E = 320000
E_TOT = E + N
NC, NS, LANES = 2, 16, 16
NW = NC * NS
MCW = 128
MC_T = 27
E_T = NW * MC_T * MCW
E_PAD = 3 * E_T
N_ACC = 10016
_KB = 3

_mesh = plsc.VectorSubcoreMesh(core_axis_name="c", subcore_axis_name="s")
_sc_params = pltpu.CompilerParams(use_tc_tiling_on_sc=False)


def _make_gather(d, mc, dt=jnp.float32, pack=1):
    rpc = MCW // pack

    def chunk_batch(tab_l, tab_r, out, vidx_l, vidx_r, vrow_l, vrow_r,
                    gsem, wsem, wid, j0, nb):
        gs = []
        for b in range(nb):
            gs.append(pltpu.async_copy(
                tab_l.at[vidx_l.at[j0 + b]], vrow_l.at[b], gsem))
            gs.append(pltpu.async_copy(
                tab_r.at[vidx_r.at[j0 + b]], vrow_r.at[b], gsem))
        for gd in gs:
            gd.wait()
        ws = []
        for b in range(nb):
            row0 = (wid * mc + j0 + b) * rpc
            for q in range(pack):
                ws.append(pltpu.async_copy(
                    vrow_l.at[b, pl.ds(q * rpc, rpc)],
                    out.at[pl.ds(row0, rpc), pl.ds(q * 2 * d, d)], wsem))
                ws.append(pltpu.async_copy(
                    vrow_r.at[b, pl.ds(q * rpc, rpc)],
                    out.at[pl.ds(row0, rpc), pl.ds(q * 2 * d + d, d)], wsem))
        for w in ws:
            w.wait()

    def body(tab_l, tab_r, idx_l, idx_r, out, vidx_l, vidx_r,
             vrow_l, vrow_r, gsem, wsem):
        cid = lax.axis_index("c")
        sid = lax.axis_index("s")
        wid = sid * NC + cid
        pltpu.sync_copy(idx_l.at[wid], vidx_l)
        pltpu.sync_copy(idx_r.at[wid], vidx_r)

        def step(g, _):
            chunk_batch(tab_l, tab_r, out, vidx_l, vidx_r, vrow_l, vrow_r,
                        gsem, wsem, wid, g * _KB, _KB)
            return 0

        lax.fori_loop(0, mc // _KB, step, 0)
        rem = mc % _KB
        if rem:
            chunk_batch(tab_l, tab_r, out, vidx_l, vidx_r, vrow_l, vrow_r,
                        gsem, wsem, wid, mc - rem, rem)

    return pl.kernel(
        body,
        out_type=jax.ShapeDtypeStruct((NW * mc * MCW // pack, 128), dt),
        mesh=_mesh,
        scratch_types=[
            pltpu.VMEM((mc, MCW), jnp.int32),
            pltpu.VMEM((mc, MCW), jnp.int32),
            pltpu.VMEM((_KB, MCW, d), dt),
            pltpu.VMEM((_KB, MCW, d), dt),
            pltpu.SemaphoreType.DMA,
            pltpu.SemaphoreType.DMA,
        ],
        compiler_params=_sc_params,
    )


def _make_scatter(dacc, mc, pack=1):
    rpc = MCW // pack

    def chunk_batch(vals, acc, vidx, vbuf, lsem, ssem, wid, j0, nb):
        ls = []
        for b in range(nb):
            row0 = (wid * mc + j0 + b) * rpc
            for q in range(pack):
                ls.append(pltpu.async_copy(
                    vals.at[pl.ds(row0, rpc), pl.ds(q * dacc, dacc)],
                    vbuf.at[b, pl.ds(q * rpc, rpc)], lsem))
        for ld in ls:
            ld.wait()
        ss = []
        for b in range(nb):
            ss.append(pltpu.async_copy(
                vbuf.at[b], acc.at[vidx.at[j0 + b]], ssem, add=True))
        for s in ss:
            s.wait()

    def body(vals, idx, zero, part, vidx, vbuf, acc, lsem, ssem):
        cid = lax.axis_index("c")
        sid = lax.axis_index("s")
        wid = sid * NC + cid

        @pl.when(sid == 0)
        def _init():
            pltpu.sync_copy(zero, acc)

        plsc.subcore_barrier()
        pltpu.sync_copy(idx.at[wid], vidx)

        def step(g, _):
            chunk_batch(vals, acc, vidx, vbuf, lsem, ssem, wid, g * _KB, _KB)
            return 0

        lax.fori_loop(0, mc // _KB, step, 0)
        rem = mc % _KB
        if rem:
            chunk_batch(vals, acc, vidx, vbuf, lsem, ssem, wid, mc - rem, rem)
        plsc.subcore_barrier()

        @pl.when(sid == 0)
        def _emit():
            pltpu.sync_copy(acc, part.at[cid])

    return pl.kernel(
        body,
        out_type=jax.ShapeDtypeStruct((NC, N_ACC, dacc), jnp.float32),
        mesh=_mesh,
        scratch_types=[
            pltpu.VMEM((mc, MCW), jnp.int32),
            pltpu.VMEM((_KB, MCW, dacc), jnp.float32),
            pltpu.VMEM_SHARED((N_ACC, dacc), jnp.float32),
            pltpu.SemaphoreType.DMA,
            pltpu.SemaphoreType.DMA,
        ],
        compiler_params=_sc_params,
    )


_BF = jnp.float32
_gathers = {64: _make_gather(64, MC_T, _BF),
            16: _make_gather(16, MC_T, pack=4)}
_scatters = {80: _make_scatter(80, MC_T),
             32: _make_scatter(32, MC_T, pack=4)}


def _mm2_body(dt, h_ref, wl_ref, wr_ref, xl_ref, xr_ref):
    h = h_ref[...]
    xl_ref[...] = jnp.dot(
        h, wl_ref[...], preferred_element_type=jnp.float32).astype(dt)
    xr_ref[...] = jnp.dot(
        h, wr_ref[...], preferred_element_type=jnp.float32).astype(dt)


def _mm2(h, wl, wr, dt=jnp.float32, blk=2000):
    n, k = h.shape
    m = wl.shape[1]
    grid = n // blk
    return pl.pallas_call(
        functools.partial(_mm2_body, dt),
        grid=(grid,),
        in_specs=[
            pl.BlockSpec((blk, k), lambda i: (i, 0)),
            pl.BlockSpec((k, m), lambda i: (0, 0)),
            pl.BlockSpec((k, m), lambda i: (0, 0)),
        ],
        out_specs=[
            pl.BlockSpec((blk, m), lambda i: (i, 0)),
            pl.BlockSpec((blk, m), lambda i: (i, 0)),
        ],
        out_shape=[jax.ShapeDtypeStruct((n, m), dt),
                   jax.ShapeDtypeStruct((n, m), dt)],
    )(h, wl, wr)


def _edge_body(heads, ch, ge_ref, att_ref, vout_ref):
    d = heads * ch
    ge = ge_ref[...].astype(jnp.float32)
    xl = ge[:, 0:d]
    z = xl + ge[:, d:2 * d]
    e = jnp.maximum(z, 0.2 * z)
    t = e * att_ref[...]
    r = lax.broadcasted_iota(jnp.int32, (d, d), 0) // ch
    c = lax.broadcasted_iota(jnp.int32, (d, d), 1) // ch
    sdup = (r == c).astype(jnp.float32)
    a_dup = jnp.exp(jnp.dot(t, sdup, preferred_element_type=jnp.float32))
    vm = xl * a_dup
    r8 = lax.broadcasted_iota(jnp.int32, (d, 16), 0) // ch
    c8 = lax.broadcasted_iota(jnp.int32, (d, 16), 1)
    s8 = jnp.logical_and(r8 == c8, c8 < heads).astype(jnp.float32)
    va = jnp.exp(jnp.dot(t, s8, preferred_element_type=jnp.float32))
    blk = vout_ref.shape[0]
    pad = 128 - 16 - d
    vout_ref[...] = jnp.concatenate(
        [va, vm, jnp.zeros((blk, pad), jnp.float32)], axis=1)


def _edge_body_p4(ge_ref, att_ref, vout_ref):
    ge = ge_ref[...]
    att = att_ref[...]
    ones = jnp.ones((16, 16), jnp.float32)
    pieces = []
    for q in range(4):
        xl = ge[:, 32 * q:32 * q + 16]
        z = xl + ge[:, 32 * q + 16:32 * q + 32]
        e = jnp.maximum(z, 0.2 * z)
        t = e * att
        a_dup = jnp.exp(jnp.dot(t, ones, preferred_element_type=jnp.float32))
        pieces.append(a_dup)
        pieces.append(xl * a_dup)
    vout_ref[...] = jnp.concatenate(pieces, axis=1)


def _edge_compute(ge, att_row, heads, ch, blk=4096):
    d = heads * ch
    rows = ge.shape[0]
    if d == 16:
        body = _edge_body_p4
        blk = 1024
    else:
        body = functools.partial(_edge_body, heads, ch)
    grid = rows // blk
    return pl.pallas_call(
        body,
        grid=(grid,),
        in_specs=[
            pl.BlockSpec((blk, 128), lambda i: (i, 0)),
            pl.BlockSpec((1, d), lambda i: (0, 0)),
        ],
        out_specs=pl.BlockSpec((blk, 128), lambda i: (i, 0)),
        out_shape=jax.ShapeDtypeStruct((rows, 128), jnp.float32),
    )(ge, att_row)


def _comb_body(p0_ref, p1_ref, p2_ref, p3_ref, p4_ref, p5_ref,
               b_ref, wl_ref, wr_ref, xl_ref, xr_ref):
    d = p0_ref.shape[1] - 16
    ch = d // 8
    p = (p0_ref[...] + p1_ref[...] + p2_ref[...] + p3_ref[...]
         + p4_ref[...] + p5_ref[...])
    s16 = p[:, 0:16]
    r = lax.broadcasted_iota(jnp.int32, (16, d), 0)
    c = lax.broadcasted_iota(jnp.int32, (16, d), 1) // ch
    rep = (r == c).astype(jnp.float32)
    sdup = jnp.dot(s16, rep, preferred_element_type=jnp.float32)
    o = p[:, 16:16 + d] / (sdup + 1e-16) + b_ref[...]
    h = jnp.where(o > 0, o, jnp.exp(jnp.minimum(o, 0.0)) - 1.0)
    dt = xl_ref.dtype
    xl_ref[...] = jnp.dot(
        h, wl_ref[...], preferred_element_type=jnp.float32).astype(dt)
    xr_ref[...] = jnp.dot(
        h, wr_ref[...], preferred_element_type=jnp.float32).astype(dt)


def _combine_next(paccs, b_row, wl, wr, dt=jnp.float32, blk=2000):
    dacc = paccs[0].shape[2]
    d = dacc - 16
    m = wl.shape[1]
    grid = N // blk
    parts = [p[c, :N] for p in paccs for c in range(NC)]
    return pl.pallas_call(
        _comb_body,
        grid=(grid,),
        in_specs=[pl.BlockSpec((blk, dacc), lambda i: (i, 0))] * 6 + [
            pl.BlockSpec((1, d), lambda i: (0, 0)),
            pl.BlockSpec((d, m), lambda i: (0, 0)),
            pl.BlockSpec((d, m), lambda i: (0, 0)),
        ],
        out_specs=[
            pl.BlockSpec((blk, m), lambda i: (i, 0)),
            pl.BlockSpec((blk, m), lambda i: (i, 0)),
        ],
        out_shape=[jax.ShapeDtypeStruct((N, m), dt),
                   jax.ShapeDtypeStruct((N, m), dt)],
    )(*parts, b_row, wl, wr)


def _final_body(p0_ref, p1_ref, p2_ref, p3_ref, p4_ref, p5_ref,
                b_ref, out_ref, ls_ref):
    p = (p0_ref[...] + p1_ref[...] + p2_ref[...] + p3_ref[...]
         + p4_ref[...] + p5_ref[...])
    s16 = p[:, 0:16]
    sel = (lax.broadcasted_iota(jnp.int32, (16, 16), 0) == 0).astype(jnp.float32)
    s = jnp.dot(s16, sel, preferred_element_type=jnp.float32)
    o = p[:, 16:32] / (s + 1e-16) + b_ref[...]
    out_ref[...] = o
    mx = jnp.max(o, axis=1, keepdims=True)
    ls_ref[...] = (o - mx) - jnp.log(jnp.sum(jnp.exp(o - mx), axis=1,
                                             keepdims=True))


def _final(paccs, b_row, blk=2000):
    grid = N // blk
    parts = [p[c, :N] for p in paccs for c in range(NC)]
    return pl.pallas_call(
        _final_body,
        grid=(grid,),
        in_specs=[pl.BlockSpec((blk, 32), lambda i: (i, 0))] * 6
        + [pl.BlockSpec((1, 16), lambda i: (0, 0))],
        out_specs=[pl.BlockSpec((blk, 16), lambda i: (i, 0))] * 2,
        out_shape=[jax.ShapeDtypeStruct((N, 16), jnp.float32),
                   jax.ShapeDtypeStruct((N, 16), jnp.float32)],
    )(*parts, b_row)


def _layer(xl, xr, idxs, att, heads, ch, zero):
    d = heads * ch
    dacc = 16 + d if d != 64 else 80
    att_row = att.reshape(1, d)
    ges = [_gathers[d](xl, xr, src, dst) for (src, dst, _) in idxs]
    vos = [_edge_compute(ge, att_row, heads, ch) for ge in ges]
    return [_scatters[dacc](vo, ds, zero)
            for vo, (_, _, ds) in zip(vos, idxs)]


def kernel(x, edge_index, W1l, W1r, att1, b1, W2l, W2r, att2, b2,
           W3l, W3r, att3, b3):
    loop = jnp.arange(N, dtype=jnp.int32)
    padz = jnp.zeros((E_PAD - E_TOT,), dtype=jnp.int32)
    src = jnp.concatenate([edge_index[0], loop, padz])
    dst = jnp.concatenate([edge_index[1], loop, padz])
    dst_s = jnp.concatenate([edge_index[1], loop,
                             jnp.full((E_PAD - E_TOT,), N, jnp.int32)])
    idxs = [
        (src[t * E_T:(t + 1) * E_T].reshape(NW, MC_T, MCW),
         dst[t * E_T:(t + 1) * E_T].reshape(NW, MC_T, MCW),
         dst_s[t * E_T:(t + 1) * E_T].reshape(NW, MC_T, MCW))
        for t in range(3)
    ]
    zero80 = jnp.zeros((N_ACC, 80), jnp.float32)
    zero32 = jnp.zeros((N_ACC, 32), jnp.float32)

    xl1, xr1 = _mm2(x, W1l, W1r, dt=_BF)
    paccs = _layer(xl1, xr1, idxs, att1, 8, 8, zero80)
    xl2, xr2 = _combine_next(paccs, b1.reshape(1, 64), W2l, W2r, dt=_BF)
    paccs = _layer(xl2, xr2, idxs, att2, 8, 8, zero80)
    xl3, xr3 = _combine_next(paccs, b2.reshape(1, 64), W3l, W3r)
    paccs = _layer(xl3, xr3, idxs, att3, 1, 16, zero32)
    out, ls = _final(paccs, b3.reshape(1, 16))
    return (out, ls)


# speedup vs baseline: 1.0860x; 1.0860x over previous
import functools

import jax
import jax.numpy as jnp
from jax import lax
from jax.experimental import pallas as pl
from jax.experimental.pallas import tpu as pltpu
from jax.experimental.pallas import tpu_sc as plsc

N = 10000
E = 320000
E_TOT = E + N
NC, NS, LANES = 2, 16, 16
NW = NC * NS
MCW = 128
MC_A, MC_B = 41, 40
E_A = NW * MC_A * MCW
E_B = NW * MC_B * MCW
E_PAD = E_A + E_B
N_ACC = 10016
_KB = 3

_mesh = plsc.VectorSubcoreMesh(core_axis_name="c", subcore_axis_name="s")
_sc_params = pltpu.CompilerParams(use_tc_tiling_on_sc=False)


def _make_gather(d, mc, dt=jnp.float32, pack=1):
    rpc = MCW // pack

    def chunk_batch(tab_l, tab_r, out, vidx_l, vidx_r, vrow_l, vrow_r,
                    gsem, wsem, wid, j0, nb):
        gs = []
        for b in range(nb):
            gs.append(pltpu.async_copy(
                tab_l.at[vidx_l.at[j0 + b]], vrow_l.at[b], gsem))
            gs.append(pltpu.async_copy(
                tab_r.at[vidx_r.at[j0 + b]], vrow_r.at[b], gsem))
        for gd in gs:
            gd.wait()
        ws = []
        for b in range(nb):
            row0 = (wid * mc + j0 + b) * rpc
            for q in range(pack):
                ws.append(pltpu.async_copy(
                    vrow_l.at[b, pl.ds(q * rpc, rpc)],
                    out.at[pl.ds(row0, rpc), pl.ds(q * 2 * d, d)], wsem))
                ws.append(pltpu.async_copy(
                    vrow_r.at[b, pl.ds(q * rpc, rpc)],
                    out.at[pl.ds(row0, rpc), pl.ds(q * 2 * d + d, d)], wsem))
        for w in ws:
            w.wait()

    def body(tab_l, tab_r, idx_l, idx_r, out, vidx_l, vidx_r,
             vrow_l, vrow_r, gsem, wsem):
        cid = lax.axis_index("c")
        sid = lax.axis_index("s")
        wid = sid * NC + cid
        pltpu.sync_copy(idx_l.at[wid], vidx_l)
        pltpu.sync_copy(idx_r.at[wid], vidx_r)

        def step(g, _):
            chunk_batch(tab_l, tab_r, out, vidx_l, vidx_r, vrow_l, vrow_r,
                        gsem, wsem, wid, g * _KB, _KB)
            return 0

        lax.fori_loop(0, mc // _KB, step, 0)
        rem = mc % _KB
        if rem:
            chunk_batch(tab_l, tab_r, out, vidx_l, vidx_r, vrow_l, vrow_r,
                        gsem, wsem, wid, mc - rem, rem)

    return pl.kernel(
        body,
        out_type=jax.ShapeDtypeStruct((NW * mc * MCW // pack, 128), dt),
        mesh=_mesh,
        scratch_types=[
            pltpu.VMEM((mc, MCW), jnp.int32),
            pltpu.VMEM((mc, MCW), jnp.int32),
            pltpu.VMEM((_KB, MCW, d), dt),
            pltpu.VMEM((_KB, MCW, d), dt),
            pltpu.SemaphoreType.DMA,
            pltpu.SemaphoreType.DMA,
        ],
        compiler_params=_sc_params,
    )


def _make_scatter(dacc, mc, pack=1):
    rpc = MCW // pack

    def chunk_batch(vals, acc, vidx, vbuf, lsem, ssem, wid, j0, nb):
        ls = []
        for b in range(nb):
            row0 = (wid * mc + j0 + b) * rpc
            for q in range(pack):
                ls.append(pltpu.async_copy(
                    vals.at[pl.ds(row0, rpc), pl.ds(q * dacc, dacc)],
                    vbuf.at[b, pl.ds(q * rpc, rpc)], lsem))
        for ld in ls:
            ld.wait()
        ss = []
        for b in range(nb):
            ss.append(pltpu.async_copy(
                vbuf.at[b], acc.at[vidx.at[j0 + b]], ssem, add=True))
        for s in ss:
            s.wait()

    def body(vals, idx, zero, part, vidx, vbuf, acc, lsem, ssem):
        cid = lax.axis_index("c")
        sid = lax.axis_index("s")
        wid = sid * NC + cid

        @pl.when(sid == 0)
        def _init():
            pltpu.sync_copy(zero, acc)

        plsc.subcore_barrier()
        pltpu.sync_copy(idx.at[wid], vidx)

        def step(g, _):
            chunk_batch(vals, acc, vidx, vbuf, lsem, ssem, wid, g * _KB, _KB)
            return 0

        lax.fori_loop(0, mc // _KB, step, 0)
        rem = mc % _KB
        if rem:
            chunk_batch(vals, acc, vidx, vbuf, lsem, ssem, wid, mc - rem, rem)
        plsc.subcore_barrier()

        @pl.when(sid == 0)
        def _emit():
            pltpu.sync_copy(acc, part.at[cid])

    return pl.kernel(
        body,
        out_type=jax.ShapeDtypeStruct((NC, N_ACC, dacc), jnp.float32),
        mesh=_mesh,
        scratch_types=[
            pltpu.VMEM((mc, MCW), jnp.int32),
            pltpu.VMEM((_KB, MCW, dacc), jnp.float32),
            pltpu.VMEM_SHARED((N_ACC, dacc), jnp.float32),
            pltpu.SemaphoreType.DMA,
            pltpu.SemaphoreType.DMA,
        ],
        compiler_params=_sc_params,
    )


_BF = jnp.float32
_gathers = {(64, MC_A): _make_gather(64, MC_A, _BF),
            (64, MC_B): _make_gather(64, MC_B, _BF),
            (16, MC_A): _make_gather(16, MC_A, pack=4),
            (16, MC_B): _make_gather(16, MC_B, pack=4)}
_scatters = {(80, MC_A): _make_scatter(80, MC_A),
             (80, MC_B): _make_scatter(80, MC_B),
             (32, MC_A): _make_scatter(32, MC_A, pack=4),
             (32, MC_B): _make_scatter(32, MC_B, pack=4)}


def _mm2_body(dt, h_ref, wl_ref, wr_ref, xl_ref, xr_ref):
    h = h_ref[...]
    xl_ref[...] = jnp.dot(
        h, wl_ref[...], preferred_element_type=jnp.float32).astype(dt)
    xr_ref[...] = jnp.dot(
        h, wr_ref[...], preferred_element_type=jnp.float32).astype(dt)


def _mm2(h, wl, wr, dt=jnp.float32, blk=2000):
    n, k = h.shape
    m = wl.shape[1]
    grid = n // blk
    return pl.pallas_call(
        functools.partial(_mm2_body, dt),
        grid=(grid,),
        in_specs=[
            pl.BlockSpec((blk, k), lambda i: (i, 0)),
            pl.BlockSpec((k, m), lambda i: (0, 0)),
            pl.BlockSpec((k, m), lambda i: (0, 0)),
        ],
        out_specs=[
            pl.BlockSpec((blk, m), lambda i: (i, 0)),
            pl.BlockSpec((blk, m), lambda i: (i, 0)),
        ],
        out_shape=[jax.ShapeDtypeStruct((n, m), dt),
                   jax.ShapeDtypeStruct((n, m), dt)],
    )(h, wl, wr)


def _edge_body(heads, ch, ge_ref, att_ref, vout_ref):
    d = heads * ch
    ge = ge_ref[...].astype(jnp.float32)
    xl = ge[:, 0:d]
    z = xl + ge[:, d:2 * d]
    e = jnp.maximum(z, 0.2 * z)
    t = e * att_ref[...]
    r = lax.broadcasted_iota(jnp.int32, (d, d), 0) // ch
    c = lax.broadcasted_iota(jnp.int32, (d, d), 1) // ch
    sdup = (r == c).astype(jnp.float32)
    a_dup = jnp.exp(jnp.dot(t, sdup, preferred_element_type=jnp.float32))
    vm = xl * a_dup
    r8 = lax.broadcasted_iota(jnp.int32, (d, 16), 0) // ch
    c8 = lax.broadcasted_iota(jnp.int32, (d, 16), 1)
    s8 = jnp.logical_and(r8 == c8, c8 < heads).astype(jnp.float32)
    va = jnp.exp(jnp.dot(t, s8, preferred_element_type=jnp.float32))
    blk = vout_ref.shape[0]
    pad = 128 - 16 - d
    vout_ref[...] = jnp.concatenate(
        [va, vm, jnp.zeros((blk, pad), jnp.float32)], axis=1)


def _edge_body_p4(ge_ref, att_ref, vout_ref):
    ge = ge_ref[...]
    att = att_ref[...]
    ones = jnp.ones((16, 16), jnp.float32)
    pieces = []
    for q in range(4):
        xl = ge[:, 32 * q:32 * q + 16]
        z = xl + ge[:, 32 * q + 16:32 * q + 32]
        e = jnp.maximum(z, 0.2 * z)
        t = e * att
        a_dup = jnp.exp(jnp.dot(t, ones, preferred_element_type=jnp.float32))
        pieces.append(a_dup)
        pieces.append(xl * a_dup)
    vout_ref[...] = jnp.concatenate(pieces, axis=1)


def _edge_compute(ge, att_row, heads, ch, blk=4096):
    d = heads * ch
    rows = ge.shape[0]
    if d == 16:
        body = _edge_body_p4
        blk = 1024
    else:
        body = functools.partial(_edge_body, heads, ch)
    grid = rows // blk
    return pl.pallas_call(
        body,
        grid=(grid,),
        in_specs=[
            pl.BlockSpec((blk, 128), lambda i: (i, 0)),
            pl.BlockSpec((1, d), lambda i: (0, 0)),
        ],
        out_specs=pl.BlockSpec((blk, 128), lambda i: (i, 0)),
        out_shape=jax.ShapeDtypeStruct((rows, 128), jnp.float32),
    )(ge, att_row)


def _comb_body(p0_ref, p1_ref, p2_ref, p3_ref,
               b_ref, wl_ref, wr_ref, xl_ref, xr_ref):
    d = p0_ref.shape[2] - 16
    ch = d // 8
    p = p0_ref[0] + p1_ref[0] + p2_ref[0] + p3_ref[0]
    s16 = p[:, 0:16]
    r = lax.broadcasted_iota(jnp.int32, (16, d), 0)
    c = lax.broadcasted_iota(jnp.int32, (16, d), 1) // ch
    rep = (r == c).astype(jnp.float32)
    sdup = jnp.dot(s16, rep, preferred_element_type=jnp.float32)
    o = p[:, 16:16 + d] / (sdup + 1e-16) + b_ref[...]
    h = jnp.where(o > 0, o, jnp.exp(jnp.minimum(o, 0.0)) - 1.0)
    dt = xl_ref.dtype
    xl_ref[...] = jnp.dot(
        h, wl_ref[...], preferred_element_type=jnp.float32).astype(dt)
    xr_ref[...] = jnp.dot(
        h, wr_ref[...], preferred_element_type=jnp.float32).astype(dt)


def _combine_next(paccs, b_row, wl, wr, dt=jnp.float32, blk=2000):
    dacc = paccs[0].shape[2]
    d = dacc - 16
    m = wl.shape[1]
    grid = N // blk
    pspecs = [pl.BlockSpec((1, blk, dacc), lambda i, c=c: (c, i, 0))
              for _ in paccs for c in range(NC)]
    return pl.pallas_call(
        _comb_body,
        grid=(grid,),
        in_specs=pspecs + [
            pl.BlockSpec((1, d), lambda i: (0, 0)),
            pl.BlockSpec((d, m), lambda i: (0, 0)),
            pl.BlockSpec((d, m), lambda i: (0, 0)),
        ],
        out_specs=[
            pl.BlockSpec((blk, m), lambda i: (i, 0)),
            pl.BlockSpec((blk, m), lambda i: (i, 0)),
        ],
        out_shape=[jax.ShapeDtypeStruct((N, m), dt),
                   jax.ShapeDtypeStruct((N, m), dt)],
    )(paccs[0], paccs[0], paccs[1], paccs[1], b_row, wl, wr)


def _final_body(p0_ref, p1_ref, p2_ref, p3_ref, b_ref, out_ref, ls_ref):
    p = p0_ref[0] + p1_ref[0] + p2_ref[0] + p3_ref[0]
    s16 = p[:, 0:16]
    sel = (lax.broadcasted_iota(jnp.int32, (16, 16), 0) == 0).astype(jnp.float32)
    s = jnp.dot(s16, sel, preferred_element_type=jnp.float32)
    o = p[:, 16:32] / (s + 1e-16) + b_ref[...]
    out_ref[...] = o
    mx = jnp.max(o, axis=1, keepdims=True)
    ls_ref[...] = (o - mx) - jnp.log(jnp.sum(jnp.exp(o - mx), axis=1,
                                             keepdims=True))


def _final(paccs, b_row, blk=2000):
    grid = N // blk
    pspecs = [pl.BlockSpec((1, blk, 32), lambda i, c=c: (c, i, 0))
              for _ in paccs for c in range(NC)]
    return pl.pallas_call(
        _final_body,
        grid=(grid,),
        in_specs=pspecs + [pl.BlockSpec((1, 16), lambda i: (0, 0))],
        out_specs=[pl.BlockSpec((blk, 16), lambda i: (i, 0))] * 2,
        out_shape=[jax.ShapeDtypeStruct((N, 16), jnp.float32),
                   jax.ShapeDtypeStruct((N, 16), jnp.float32)],
    )(paccs[0], paccs[0], paccs[1], paccs[1], b_row)


def _layer(xl, xr, idxs, att, heads, ch, zero):
    d = heads * ch
    dacc = 16 + d if d != 64 else 80
    att_row = att.reshape(1, d)
    mcs = (MC_A, MC_B)
    ges = [_gathers[(d, mc)](xl, xr, src, dst)
           for mc, (src, dst, _) in zip(mcs, idxs)]
    vos = [_edge_compute(ge, att_row, heads, ch) for ge in ges]
    return [_scatters[(dacc, mc)](vo, ds, zero)
            for mc, (vo, (_, _, ds)) in zip(mcs, zip(vos, idxs))]


def kernel(x, edge_index, W1l, W1r, att1, b1, W2l, W2r, att2, b2,
           W3l, W3r, att3, b3):
    loop = jnp.arange(N, dtype=jnp.int32)
    padz = jnp.zeros((E_PAD - E_TOT,), dtype=jnp.int32)
    src = jnp.concatenate([edge_index[0], loop, padz])
    dst = jnp.concatenate([edge_index[1], loop, padz])
    dst_s = jnp.concatenate([edge_index[1], loop,
                             jnp.full((E_PAD - E_TOT,), N, jnp.int32)])
    idxs = [
        (src[:E_A].reshape(NW, MC_A, MCW), dst[:E_A].reshape(NW, MC_A, MCW),
         dst_s[:E_A].reshape(NW, MC_A, MCW)),
        (src[E_A:].reshape(NW, MC_B, MCW), dst[E_A:].reshape(NW, MC_B, MCW),
         dst_s[E_A:].reshape(NW, MC_B, MCW)),
    ]
    zero80 = jnp.zeros((N_ACC, 80), jnp.float32)
    zero32 = jnp.zeros((N_ACC, 32), jnp.float32)

    xl1, xr1 = _mm2(x, W1l, W1r, dt=_BF)
    paccs = _layer(xl1, xr1, idxs, att1, 8, 8, zero80)
    xl2, xr2 = _combine_next(paccs, b1.reshape(1, 64), W2l, W2r, dt=_BF)
    paccs = _layer(xl2, xr2, idxs, att2, 8, 8, zero80)
    xl3, xr3 = _combine_next(paccs, b2.reshape(1, 64), W3l, W3r)
    paccs = _layer(xl3, xr3, idxs, att3, 1, 16, zero32)
    out, ls = _final(paccs, b3.reshape(1, 16))
    return (out, ls)


# speedup vs baseline: 1.0904x; 1.0041x over previous
import functools

import jax
import jax.numpy as jnp
from jax import lax
from jax.experimental import pallas as pl
from jax.experimental.pallas import tpu as pltpu
from jax.experimental.pallas import tpu_sc as plsc

N = 10000
E = 320000
E_TOT = E + N
NC, NS, LANES = 2, 16, 16
NW = NC * NS
MCW = 128
MC_A, MC_B = 41, 40
E_A = NW * MC_A * MCW
E_B = NW * MC_B * MCW
E_PAD = E_A + E_B
N_ACC = 10016
_KB = 3

_mesh = plsc.VectorSubcoreMesh(core_axis_name="c", subcore_axis_name="s")
_sc_params = pltpu.CompilerParams(use_tc_tiling_on_sc=False)


def _make_gather(d, mc, dt=jnp.float32, pack=1):
    rpc = MCW // pack

    def chunk_batch(tab_l, tab_r, out, vidx_l, vidx_r, vrow_l, vrow_r,
                    gsem, wsem, wid, j0, nb):
        gs = []
        for b in range(nb):
            gs.append(pltpu.async_copy(
                tab_l.at[vidx_l.at[j0 + b]], vrow_l.at[b], gsem))
            gs.append(pltpu.async_copy(
                tab_r.at[vidx_r.at[j0 + b]], vrow_r.at[b], gsem))
        for gd in gs:
            gd.wait()
        ws = []
        for b in range(nb):
            row0 = (wid * mc + j0 + b) * rpc
            for q in range(pack):
                ws.append(pltpu.async_copy(
                    vrow_l.at[b, pl.ds(q * rpc, rpc)],
                    out.at[pl.ds(row0, rpc), pl.ds(q * 2 * d, d)], wsem))
                ws.append(pltpu.async_copy(
                    vrow_r.at[b, pl.ds(q * rpc, rpc)],
                    out.at[pl.ds(row0, rpc), pl.ds(q * 2 * d + d, d)], wsem))
        for w in ws:
            w.wait()

    def body(tab_l, tab_r, idx_l, idx_r, out, vidx_l, vidx_r,
             vrow_l, vrow_r, gsem, wsem):
        cid = lax.axis_index("c")
        sid = lax.axis_index("s")
        wid = sid * NC + cid
        pltpu.sync_copy(idx_l.at[wid], vidx_l)
        pltpu.sync_copy(idx_r.at[wid], vidx_r)

        def step(g, _):
            chunk_batch(tab_l, tab_r, out, vidx_l, vidx_r, vrow_l, vrow_r,
                        gsem, wsem, wid, g * _KB, _KB)
            return 0

        lax.fori_loop(0, mc // _KB, step, 0)
        rem = mc % _KB
        if rem:
            chunk_batch(tab_l, tab_r, out, vidx_l, vidx_r, vrow_l, vrow_r,
                        gsem, wsem, wid, mc - rem, rem)

    return pl.kernel(
        body,
        out_type=jax.ShapeDtypeStruct((NW * mc * MCW // pack, 128), dt),
        mesh=_mesh,
        scratch_types=[
            pltpu.VMEM((mc, MCW), jnp.int32),
            pltpu.VMEM((mc, MCW), jnp.int32),
            pltpu.VMEM((_KB, MCW, d), dt),
            pltpu.VMEM((_KB, MCW, d), dt),
            pltpu.SemaphoreType.DMA,
            pltpu.SemaphoreType.DMA,
        ],
        compiler_params=_sc_params,
    )


def _make_scatter(dacc, mc, pack=1):
    rpc = MCW // pack

    def chunk_batch(vals, acc, vidx, vbuf, lsem, ssem, wid, j0, nb):
        ls = []
        for b in range(nb):
            row0 = (wid * mc + j0 + b) * rpc
            for q in range(pack):
                ls.append(pltpu.async_copy(
                    vals.at[pl.ds(row0, rpc), pl.ds(q * dacc, dacc)],
                    vbuf.at[b, pl.ds(q * rpc, rpc)], lsem))
        for ld in ls:
            ld.wait()
        ss = []
        for b in range(nb):
            ss.append(pltpu.async_copy(
                vbuf.at[b], acc.at[vidx.at[j0 + b]], ssem, add=True))
        for s in ss:
            s.wait()

    def body(vals, idx, zero, part, vidx, vbuf, acc, lsem, ssem):
        cid = lax.axis_index("c")
        sid = lax.axis_index("s")
        wid = sid * NC + cid

        @pl.when(sid == 0)
        def _init():
            pltpu.sync_copy(zero, acc)

        plsc.subcore_barrier()
        pltpu.sync_copy(idx.at[wid], vidx)

        def step(g, _):
            chunk_batch(vals, acc, vidx, vbuf, lsem, ssem, wid, g * _KB, _KB)
            return 0

        lax.fori_loop(0, mc // _KB, step, 0)
        rem = mc % _KB
        if rem:
            chunk_batch(vals, acc, vidx, vbuf, lsem, ssem, wid, mc - rem, rem)
        plsc.subcore_barrier()

        @pl.when(sid == 0)
        def _emit():
            pltpu.sync_copy(acc, part.at[cid])

    return pl.kernel(
        body,
        out_type=jax.ShapeDtypeStruct((NC, N_ACC, dacc), jnp.float32),
        mesh=_mesh,
        scratch_types=[
            pltpu.VMEM((mc, MCW), jnp.int32),
            pltpu.VMEM((_KB, MCW, dacc), jnp.float32),
            pltpu.VMEM_SHARED((N_ACC, dacc), jnp.float32),
            pltpu.SemaphoreType.DMA,
            pltpu.SemaphoreType.DMA,
        ],
        compiler_params=_sc_params,
    )


_BF = jnp.float32
_gathers = {(64, MC_A): _make_gather(64, MC_A, _BF),
            (64, MC_B): _make_gather(64, MC_B, _BF),
            (16, MC_A): _make_gather(16, MC_A, pack=4),
            (16, MC_B): _make_gather(16, MC_B, pack=4)}
_scatters = {(80, MC_A): _make_scatter(80, MC_A),
             (80, MC_B): _make_scatter(80, MC_B),
             (32, MC_A): _make_scatter(32, MC_A, pack=4),
             (32, MC_B): _make_scatter(32, MC_B, pack=4)}


def _mm2_body(dt, h_ref, wl_ref, wr_ref, xl_ref, xr_ref):
    h = h_ref[...]
    xl_ref[...] = jnp.dot(
        h, wl_ref[...], preferred_element_type=jnp.float32).astype(dt)
    xr_ref[...] = jnp.dot(
        h, wr_ref[...], preferred_element_type=jnp.float32).astype(dt)


def _mm2(h, wl, wr, dt=jnp.float32, blk=2000):
    n, k = h.shape
    m = wl.shape[1]
    grid = n // blk
    return pl.pallas_call(
        functools.partial(_mm2_body, dt),
        grid=(grid,),
        in_specs=[
            pl.BlockSpec((blk, k), lambda i: (i, 0)),
            pl.BlockSpec((k, m), lambda i: (0, 0)),
            pl.BlockSpec((k, m), lambda i: (0, 0)),
        ],
        out_specs=[
            pl.BlockSpec((blk, m), lambda i: (i, 0)),
            pl.BlockSpec((blk, m), lambda i: (i, 0)),
        ],
        out_shape=[jax.ShapeDtypeStruct((n, m), dt),
                   jax.ShapeDtypeStruct((n, m), dt)],
    )(h, wl, wr)


def _edge_body(heads, ch, ge_ref, att_ref, vout_ref):
    d = heads * ch
    ge = ge_ref[...].astype(jnp.float32)
    xl = ge[:, 0:d]
    z = xl + ge[:, d:2 * d]
    e = jnp.maximum(z, 0.2 * z)
    t = e * att_ref[...]
    r8 = lax.broadcasted_iota(jnp.int32, (d, 16), 0) // ch
    c8 = lax.broadcasted_iota(jnp.int32, (d, 16), 1)
    s8 = jnp.logical_and(r8 == c8, c8 < heads).astype(jnp.float32)
    va = jnp.exp(jnp.dot(t, s8, preferred_element_type=jnp.float32))
    rr = lax.broadcasted_iota(jnp.int32, (16, d), 0)
    cc = lax.broadcasted_iota(jnp.int32, (16, d), 1) // ch
    rep = (rr == cc).astype(jnp.float32)
    a_dup = jnp.dot(va, rep, preferred_element_type=jnp.float32)
    vm = xl * a_dup
    blk = vout_ref.shape[0]
    pad = 128 - 16 - d
    vout_ref[...] = jnp.concatenate(
        [va, vm, jnp.zeros((blk, pad), jnp.float32)], axis=1)


def _edge_body_p4(ge_ref, att_ref, vout_ref):
    ge = ge_ref[...]
    att = att_ref[...]
    ones = jnp.ones((16, 16), jnp.float32)
    pieces = []
    for q in range(4):
        xl = ge[:, 32 * q:32 * q + 16]
        z = xl + ge[:, 32 * q + 16:32 * q + 32]
        e = jnp.maximum(z, 0.2 * z)
        t = e * att
        a_dup = jnp.exp(jnp.dot(t, ones, preferred_element_type=jnp.float32))
        pieces.append(a_dup)
        pieces.append(xl * a_dup)
    vout_ref[...] = jnp.concatenate(pieces, axis=1)


def _edge_compute(ge, att_row, heads, ch, blk=4096):
    d = heads * ch
    rows = ge.shape[0]
    if d == 16:
        body = _edge_body_p4
        blk = 1024
    else:
        body = functools.partial(_edge_body, heads, ch)
    grid = rows // blk
    return pl.pallas_call(
        body,
        grid=(grid,),
        in_specs=[
            pl.BlockSpec((blk, 128), lambda i: (i, 0)),
            pl.BlockSpec((1, d), lambda i: (0, 0)),
        ],
        out_specs=pl.BlockSpec((blk, 128), lambda i: (i, 0)),
        out_shape=jax.ShapeDtypeStruct((rows, 128), jnp.float32),
    )(ge, att_row)


def _comb_body(p0_ref, p1_ref, p2_ref, p3_ref,
               b_ref, wl_ref, wr_ref, xl_ref, xr_ref):
    d = p0_ref.shape[2] - 16
    ch = d // 8
    p = p0_ref[0] + p1_ref[0] + p2_ref[0] + p3_ref[0]
    s16 = p[:, 0:16]
    r = lax.broadcasted_iota(jnp.int32, (16, d), 0)
    c = lax.broadcasted_iota(jnp.int32, (16, d), 1) // ch
    rep = (r == c).astype(jnp.float32)
    sdup = jnp.dot(s16, rep, preferred_element_type=jnp.float32)
    o = p[:, 16:16 + d] / (sdup + 1e-16) + b_ref[...]
    h = jnp.where(o > 0, o, jnp.exp(jnp.minimum(o, 0.0)) - 1.0)
    dt = xl_ref.dtype
    xl_ref[...] = jnp.dot(
        h, wl_ref[...], preferred_element_type=jnp.float32).astype(dt)
    xr_ref[...] = jnp.dot(
        h, wr_ref[...], preferred_element_type=jnp.float32).astype(dt)


def _combine_next(paccs, b_row, wl, wr, dt=jnp.float32, blk=2000):
    dacc = paccs[0].shape[2]
    d = dacc - 16
    m = wl.shape[1]
    grid = N // blk
    pspecs = [pl.BlockSpec((1, blk, dacc), lambda i, c=c: (c, i, 0))
              for _ in paccs for c in range(NC)]
    return pl.pallas_call(
        _comb_body,
        grid=(grid,),
        in_specs=pspecs + [
            pl.BlockSpec((1, d), lambda i: (0, 0)),
            pl.BlockSpec((d, m), lambda i: (0, 0)),
            pl.BlockSpec((d, m), lambda i: (0, 0)),
        ],
        out_specs=[
            pl.BlockSpec((blk, m), lambda i: (i, 0)),
            pl.BlockSpec((blk, m), lambda i: (i, 0)),
        ],
        out_shape=[jax.ShapeDtypeStruct((N, m), dt),
                   jax.ShapeDtypeStruct((N, m), dt)],
    )(paccs[0], paccs[0], paccs[1], paccs[1], b_row, wl, wr)


def _final_body(p0_ref, p1_ref, p2_ref, p3_ref, b_ref, out_ref, ls_ref):
    p = p0_ref[0] + p1_ref[0] + p2_ref[0] + p3_ref[0]
    s16 = p[:, 0:16]
    sel = (lax.broadcasted_iota(jnp.int32, (16, 16), 0) == 0).astype(jnp.float32)
    s = jnp.dot(s16, sel, preferred_element_type=jnp.float32)
    o = p[:, 16:32] / (s + 1e-16) + b_ref[...]
    out_ref[...] = o
    mx = jnp.max(o, axis=1, keepdims=True)
    ls_ref[...] = (o - mx) - jnp.log(jnp.sum(jnp.exp(o - mx), axis=1,
                                             keepdims=True))


def _final(paccs, b_row, blk=2000):
    grid = N // blk
    pspecs = [pl.BlockSpec((1, blk, 32), lambda i, c=c: (c, i, 0))
              for _ in paccs for c in range(NC)]
    return pl.pallas_call(
        _final_body,
        grid=(grid,),
        in_specs=pspecs + [pl.BlockSpec((1, 16), lambda i: (0, 0))],
        out_specs=[pl.BlockSpec((blk, 16), lambda i: (i, 0))] * 2,
        out_shape=[jax.ShapeDtypeStruct((N, 16), jnp.float32),
                   jax.ShapeDtypeStruct((N, 16), jnp.float32)],
    )(paccs[0], paccs[0], paccs[1], paccs[1], b_row)


def _layer(xl, xr, idxs, att, heads, ch, zero):
    d = heads * ch
    dacc = 16 + d if d != 64 else 80
    att_row = att.reshape(1, d)
    mcs = (MC_A, MC_B)
    ges = [_gathers[(d, mc)](xl, xr, src, dst)
           for mc, (src, dst, _) in zip(mcs, idxs)]
    vos = [_edge_compute(ge, att_row, heads, ch) for ge in ges]
    return [_scatters[(dacc, mc)](vo, ds, zero)
            for mc, (vo, (_, _, ds)) in zip(mcs, zip(vos, idxs))]


def kernel(x, edge_index, W1l, W1r, att1, b1, W2l, W2r, att2, b2,
           W3l, W3r, att3, b3):
    loop = jnp.arange(N, dtype=jnp.int32)
    padz = jnp.zeros((E_PAD - E_TOT,), dtype=jnp.int32)
    src = jnp.concatenate([edge_index[0], loop, padz])
    dst = jnp.concatenate([edge_index[1], loop, padz])
    dst_s = jnp.concatenate([edge_index[1], loop,
                             jnp.full((E_PAD - E_TOT,), N, jnp.int32)])
    idxs = [
        (src[:E_A].reshape(NW, MC_A, MCW), dst[:E_A].reshape(NW, MC_A, MCW),
         dst_s[:E_A].reshape(NW, MC_A, MCW)),
        (src[E_A:].reshape(NW, MC_B, MCW), dst[E_A:].reshape(NW, MC_B, MCW),
         dst_s[E_A:].reshape(NW, MC_B, MCW)),
    ]
    zero80 = jnp.zeros((N_ACC, 80), jnp.float32)
    zero32 = jnp.zeros((N_ACC, 32), jnp.float32)

    xl1, xr1 = _mm2(x, W1l, W1r, dt=_BF)
    paccs = _layer(xl1, xr1, idxs, att1, 8, 8, zero80)
    xl2, xr2 = _combine_next(paccs, b1.reshape(1, 64), W2l, W2r, dt=_BF)
    paccs = _layer(xl2, xr2, idxs, att2, 8, 8, zero80)
    xl3, xr3 = _combine_next(paccs, b2.reshape(1, 64), W3l, W3r)
    paccs = _layer(xl3, xr3, idxs, att3, 1, 16, zero32)
    out, ls = _final(paccs, b3.reshape(1, 16))
    return (out, ls)
